# baseline (speedup 1.0000x reference)
import jax
import jax.numpy as jnp
from jax import lax
from jax.experimental import pallas as pl
from jax.experimental.pallas import tpu as pltpu
import jax.experimental.pallas.tpu_sc as plsc

N = 10000
E = 320000
D = 128
G = 32
F1 = 32
F2P = 32

NC = 2
NS = 16
NW = NC * NS
CH = 128
K = 80
E_PAD = NW * K * CH
ACC_ROWS = 10240
RPT = ACC_ROWS // NS
NB = 5


def _seg_body(h_hbm, edges_hbm, out_hbm,
              src_all, dst_all, bufs, zbuf, hs, acc, sem_g, sem_s):
    cid = lax.axis_index("c")
    sid = lax.axis_index("s")
    wid = cid * NS + sid
    rh = N // NS

    z16 = jnp.zeros((16,), jnp.float32)

    def zrow(i, carry):
        zbuf[i, pl.ds(0, 16)] = z16
        zbuf[i, pl.ds(16, 16)] = z16
        return carry

    lax.fori_loop(0, 128, zrow, 0)
    for q in range(RPT // 128):
        pltpu.sync_copy(zbuf, acc.at[pl.ds(sid * RPT + q * 128, 128)])

    pltpu.sync_copy(h_hbm.at[pl.ds(sid * rh, rh)], hs.at[pl.ds(sid * rh, rh)])
    pltpu.sync_copy(edges_hbm.at[0, wid], src_all)
    pltpu.sync_copy(edges_hbm.at[1, wid], dst_all)
    plsc.subcore_barrier()

    ba = [bufs.at[i] for i in range(NB)]
    bb = [bufs.at[NB + i] for i in range(NB)]

    def gather(j, b):
        pltpu.async_copy(hs.at[src_all.at[j]], b, sem_g)

    def wait_gather(j, b):
        pltpu.make_async_copy(hs.at[src_all.at[j]], b, sem_g).wait()

    def scatter(j, b):
        pltpu.async_copy(b, acc.at[dst_all.at[j]], sem_s, add=True)

    def wait_scatter(j, b):
        pltpu.make_async_copy(b, acc.at[dst_all.at[j]], sem_s).wait()

    for i in range(NB):
        gather(i, ba[i])

    def group(g, carry):
        for i in range(NB):
            wait_gather(g * NB + i, ba[i])
        for i in range(NB):
            gather((g + 1) * NB + i, bb[i])
        for i in range(NB):
            scatter(g * NB + i, ba[i])
        for i in range(NB):
            wait_gather((g + 1) * NB + i, bb[i])
        for i in range(NB):
            wait_scatter(g * NB + i, ba[i])

        @pl.when(g + 2 < K // NB)
        def _():
            for i in range(NB):
                gather((g + 2) * NB + i, ba[i])

        for i in range(NB):
            scatter((g + 1) * NB + i, bb[i])
        for i in range(NB):
            wait_scatter((g + 1) * NB + i, bb[i])
        return carry

    lax.fori_loop(0, K // (2 * NB), lambda t, c: group(t * 2, c), 0)
    plsc.subcore_barrier()

    pltpu.sync_copy(acc.at[pl.ds(sid * RPT, RPT)],
                    out_hbm.at[cid, pl.ds(sid * RPT, RPT)])


def _make_seg_kernel():
    return pl.kernel(
        _seg_body,
        out_type=jax.ShapeDtypeStruct((NC, ACC_ROWS, F1), jnp.float32),
        mesh=plsc.VectorSubcoreMesh(core_axis_name="c", subcore_axis_name="s"),
        compiler_params=pltpu.CompilerParams(use_tc_tiling_on_sc=False),
        scratch_types=[
            pltpu.VMEM((K, CH), jnp.int32),
            pltpu.VMEM((K, CH), jnp.int32),
            pltpu.VMEM((2 * NB, CH, F1), jnp.float32),
            pltpu.VMEM((128, F1), jnp.float32),
            pltpu.VMEM_SHARED((N, F1), jnp.float32),
            pltpu.VMEM_SHARED((ACC_ROWS, F1), jnp.float32),
            pltpu.SemaphoreType.DMA,
            pltpu.SemaphoreType.DMA,
        ],
    )


def _mm1_body(x_ref, w_ref, o_ref):
    o_ref[...] = jnp.dot(x_ref[...], w_ref[...],
                         preferred_element_type=jnp.float32)


def _mid_body(p_ref, b1_ref, w2_ref, o_ref):
    h = p_ref[0] + p_ref[1] + b1_ref[...]
    h = jnp.where(h > 0, h, 0.01 * h)
    w2 = jnp.concatenate(
        [w2_ref[...], jnp.zeros((F1, F2P - 20), jnp.float32)], axis=1)
    o_ref[...] = jnp.dot(h, w2, preferred_element_type=jnp.float32)


def _tail_body(p_ref, batch_ref, b2_ref, g_ref, be_ref,
               wf1_ref, bf1_ref, wf2_ref, bf2_ref, o_ref):
    pad = jnp.zeros((1, F2P - 20), jnp.float32)
    b2 = jnp.concatenate([b2_ref[...], pad], axis=1)
    gam = jnp.concatenate([g_ref[...], 1.0 + pad], axis=1)
    bet = jnp.concatenate([be_ref[...], pad], axis=1)
    h = p_ref[0, :N, :] + p_ref[1, :N, :] + b2
    mean = jnp.sum(h, axis=0, keepdims=True) * (1.0 / N)
    var = jnp.sum(h * h, axis=0, keepdims=True) * (1.0 / N) - mean * mean
    hn = (h - mean) * lax.rsqrt(var + 1e-5) * gam + bet
    hn = jnp.where(hn > 0, hn, 0.01 * hn)
    gid = lax.broadcasted_iota(jnp.int32, (N, G), 1)
    m = (batch_ref[...] == gid).astype(jnp.float32)
    pooled = lax.dot_general(m, hn, (((0,), (0,)), ((), ())),
                             preferred_element_type=jnp.float32)
    z = jnp.dot(pooled[:, :20], wf1_ref[...],
                preferred_element_type=jnp.float32) + bf1_ref[...]
    z = jnp.where(z > 0, z, 0.01 * z)
    z = jnp.dot(z, wf2_ref[...],
                preferred_element_type=jnp.float32) + bf2_ref[...]
    zmax = jnp.max(z, axis=-1, keepdims=True)
    ze = z - zmax
    o_ref[...] = ze - jnp.log(jnp.sum(jnp.exp(ze), axis=-1, keepdims=True))


def kernel(x, edge_index, batch, W1, b1, W2, b2, gamma, beta,
           Wf1, bf1, Wf2, bf2):
    ppw = (E_PAD - E) // NW
    pad_idx = jnp.broadcast_to(
        jnp.arange(ppw, dtype=jnp.int32)[None, None, :], (1, NW, ppw))
    pad2 = jnp.concatenate([pad_idx, N + pad_idx], axis=0)
    edges = jnp.concatenate(
        [edge_index.reshape(2, NW, E // NW), pad2], axis=2
    ).reshape(2, NW, K, CH)

    mm1 = pl.pallas_call(
        _mm1_body,
        grid=(10,),
        in_specs=[pl.BlockSpec((N // 10, D), lambda i: (i, 0)),
                  pl.BlockSpec((D, F1), lambda i: (0, 0))],
        out_specs=pl.BlockSpec((N // 10, F1), lambda i: (i, 0)),
        out_shape=jax.ShapeDtypeStruct((N, F1), jnp.float32),
    )
    h = mm1(x, W1)

    p1 = _make_seg_kernel()(h, edges)

    mid = pl.pallas_call(
        _mid_body,
        grid=(10,),
        in_specs=[pl.BlockSpec((2, N // 10, F1), lambda i: (0, i, 0)),
                  pl.BlockSpec((1, F1), lambda i: (0, 0)),
                  pl.BlockSpec((F1, 20), lambda i: (0, 0))],
        out_specs=pl.BlockSpec((N // 10, F2P), lambda i: (i, 0)),
        out_shape=jax.ShapeDtypeStruct((N, F2P), jnp.float32),
    )
    h2 = mid(p1, b1.reshape(1, F1), W2)

    p2 = _make_seg_kernel()(h2, edges)

    tail = pl.pallas_call(
        _tail_body,
        in_specs=[pl.BlockSpec((2, ACC_ROWS, F2P), lambda: (0, 0, 0)),
                  pl.BlockSpec((N, 1), lambda: (0, 0)),
                  pl.BlockSpec((1, 20), lambda: (0, 0)),
                  pl.BlockSpec((1, 20), lambda: (0, 0)),
                  pl.BlockSpec((1, 20), lambda: (0, 0)),
                  pl.BlockSpec((20, 10), lambda: (0, 0)),
                  pl.BlockSpec((1, 10), lambda: (0, 0)),
                  pl.BlockSpec((10, 2), lambda: (0, 0)),
                  pl.BlockSpec((1, 2), lambda: (0, 0))],
        out_specs=pl.BlockSpec((G, 2), lambda: (0, 0)),
        out_shape=jax.ShapeDtypeStruct((G, 2), jnp.float32),
    )
    return tail(p2, batch.reshape(N, 1), b2.reshape(1, 20),
                gamma.reshape(1, 20), beta.reshape(1, 20),
                Wf1, bf1.reshape(1, 10), Wf2, bf2.reshape(1, 2))


# speedup vs baseline: 1.3085x; 1.3085x over previous
import jax
import jax.numpy as jnp
from jax import lax
from jax.experimental import pallas as pl
from jax.experimental.pallas import tpu as pltpu
import jax.experimental.pallas.tpu_sc as plsc

N = 10000
E = 320000
D = 128
G = 32
F1 = 32
F2P = 32

NC = 2
NS = 16
NW = NC * NS
CH = 128
K = 80
E_PAD = NW * K * CH
ACC_ROWS = 10240
RPT = ACC_ROWS // NS
NB = 8


def _seg_body(h_hbm, src_hbm, dst_hbm, out_hbm,
              src_all, dst_all, bufs, zbuf, acc, sem_g, sem_s):
    cid = lax.axis_index("c")
    sid = lax.axis_index("s")
    wid = cid * NS + sid

    z16 = jnp.zeros((16,), jnp.float32)

    def zrow(i, carry):
        zbuf[i, pl.ds(0, 16)] = z16
        zbuf[i, pl.ds(16, 16)] = z16
        return carry

    lax.fori_loop(0, RPT, zrow, 0)
    pltpu.sync_copy(zbuf, acc.at[pl.ds(sid * RPT, RPT)])

    pltpu.sync_copy(src_hbm.at[pl.ds(wid * K, K)], src_all)
    pltpu.sync_copy(dst_hbm.at[pl.ds(wid * K, K)], dst_all)
    plsc.subcore_barrier()

    ba = [bufs.at[i] for i in range(NB)]
    bb = [bufs.at[NB + i] for i in range(NB)]

    def gather(j, b):
        pltpu.async_copy(h_hbm.at[src_all.at[j]], b, sem_g)

    def wait_gather(j, b):
        pltpu.make_async_copy(h_hbm.at[src_all.at[j]], b, sem_g).wait()

    def scatter(j, b):
        pltpu.async_copy(b, acc.at[dst_all.at[j]], sem_s, add=True)

    def wait_scatter(j, b):
        pltpu.make_async_copy(b, acc.at[dst_all.at[j]], sem_s).wait()

    for i in range(NB):
        gather(i, ba[i])

    def group(g, carry):
        for i in range(NB):
            wait_gather(g * NB + i, ba[i])
        for i in range(NB):
            gather((g + 1) * NB + i, bb[i])
        for i in range(NB):
            scatter(g * NB + i, ba[i])
        for i in range(NB):
            wait_gather((g + 1) * NB + i, bb[i])
        for i in range(NB):
            wait_scatter(g * NB + i, ba[i])

        @pl.when(g + 2 < K // NB)
        def _():
            for i in range(NB):
                gather((g + 2) * NB + i, ba[i])

        for i in range(NB):
            scatter((g + 1) * NB + i, bb[i])
        for i in range(NB):
            wait_scatter((g + 1) * NB + i, bb[i])
        return carry

    lax.fori_loop(0, K // (2 * NB), lambda t, c: group(t * 2, c), 0)
    plsc.subcore_barrier()

    pltpu.sync_copy(acc.at[pl.ds(sid * RPT, RPT)],
                    out_hbm.at[cid, pl.ds(sid * RPT, RPT)])


def _make_seg_kernel():
    return pl.kernel(
        _seg_body,
        out_type=jax.ShapeDtypeStruct((NC, ACC_ROWS, F1), jnp.float32),
        mesh=plsc.VectorSubcoreMesh(core_axis_name="c", subcore_axis_name="s"),
        compiler_params=pltpu.CompilerParams(use_tc_tiling_on_sc=False),
        scratch_types=[
            pltpu.VMEM((K, CH), jnp.int32),
            pltpu.VMEM((K, CH), jnp.int32),
            pltpu.VMEM((2 * NB, CH, F1), jnp.float32),
            pltpu.VMEM((RPT, F1), jnp.float32),
            pltpu.VMEM_SHARED((ACC_ROWS, F1), jnp.float32),
            pltpu.SemaphoreType.DMA,
            pltpu.SemaphoreType.DMA,
        ],
    )


PK = 128 // F1
HP = N // PK
PP = ACC_ROWS // PK


def _mm1_body(x_ref, w_ref, o_ref):
    xa = x_ref[...].reshape(-1, PK, D)
    parts = [jnp.dot(xa[:, j, :], w_ref[...],
                     preferred_element_type=jnp.float32) for j in range(PK)]
    o_ref[...] = jnp.concatenate(parts, axis=1)


def _mid_body(p_ref, b1_ref, o_ref):
    b1t = jnp.concatenate([b1_ref[...]] * PK, axis=1)
    h = p_ref[0, :HP, :] + p_ref[1, :HP, :] + b1t
    o_ref[...] = jnp.where(h > 0, h, 0.01 * h)


def _tail_body(p_ref, batch_ref, w2_ref, b2_ref, g_ref, be_ref,
               wf1_ref, bf1_ref, wf2_ref, bf2_ref, o_ref):
    q = p_ref[0, :HP, :] + p_ref[1, :HP, :]
    w2big = jnp.concatenate(
        [jnp.pad(w2_ref[...], ((0, 0), (20 * j, 60 - 20 * j)))
         for j in range(PK)], axis=0)
    b2t = jnp.concatenate([b2_ref[...]] * PK, axis=1)
    hh = jnp.dot(q, w2big, preferred_element_type=jnp.float32) + b2t
    s = jnp.sum(hh, axis=0, keepdims=True)
    mean = (s[:, :20] + s[:, 20:40] + s[:, 40:60] + s[:, 60:]) * (1.0 / N)
    hc = hh - jnp.concatenate([mean] * PK, axis=1)
    s2 = jnp.sum(hc * hc, axis=0, keepdims=True)
    var = (s2[:, :20] + s2[:, 20:40] + s2[:, 40:60] + s2[:, 60:]) * (1.0 / N)
    scale = lax.rsqrt(var + 1e-5) * g_ref[...]
    shift = be_ref[...] - mean * scale
    sc = jnp.concatenate([scale] * PK, axis=1)
    sh = jnp.concatenate([shift] * PK, axis=1)
    hn = hh * sc + sh
    hn = jnp.where(hn > 0, hn, 0.01 * hn)
    bt = batch_ref[...]
    gid = lax.broadcasted_iota(jnp.int32, (HP, G), 1)
    pooled = jnp.zeros((G, 20), jnp.float32)
    for j in range(PK):
        m = (bt[:, j:j + 1] == gid).astype(jnp.float32)
        pooled = pooled + lax.dot_general(
            m, hn[:, 20 * j:20 * j + 20], (((0,), (0,)), ((), ())),
            preferred_element_type=jnp.float32)
    z = jnp.dot(pooled, wf1_ref[...],
                preferred_element_type=jnp.float32) + bf1_ref[...]
    z = jnp.where(z > 0, z, 0.01 * z)
    z = jnp.dot(z, wf2_ref[...],
                preferred_element_type=jnp.float32) + bf2_ref[...]
    zmax = jnp.max(z, axis=-1, keepdims=True)
    ze = z - zmax
    o_ref[...] = ze - jnp.log(jnp.sum(jnp.exp(ze), axis=-1, keepdims=True))


def kernel(x, edge_index, batch, W1, b1, W2, b2, gamma, beta,
           Wf1, bf1, Wf2, bf2):
    pad_pos = jnp.arange(E_PAD - E, dtype=jnp.int32)
    srcp = jnp.concatenate(
        [edge_index[0], pad_pos % N]).reshape(E_PAD // CH, CH)
    dstp = jnp.concatenate(
        [edge_index[1], N + pad_pos % (ACC_ROWS - N)]).reshape(E_PAD // CH, CH)

    mm1 = pl.pallas_call(
        _mm1_body,
        in_specs=[pl.BlockSpec((N, D), lambda: (0, 0)),
                  pl.BlockSpec((D, F1), lambda: (0, 0))],
        out_specs=pl.BlockSpec((HP, 128), lambda: (0, 0)),
        out_shape=jax.ShapeDtypeStruct((HP, 128), jnp.float32),
    )
    h = mm1(x, W1)

    seg = _make_seg_kernel()
    p1 = seg(h.reshape(N, F1), srcp, dstp)

    mid = pl.pallas_call(
        _mid_body,
        in_specs=[pl.BlockSpec((2, PP, 128), lambda: (0, 0, 0)),
                  pl.BlockSpec((1, F1), lambda: (0, 0))],
        out_specs=pl.BlockSpec((HP, 128), lambda: (0, 0)),
        out_shape=jax.ShapeDtypeStruct((HP, 128), jnp.float32),
    )
    h2 = mid(p1.reshape(NC, PP, 128), b1.reshape(1, F1))

    p2 = seg(h2.reshape(N, F1), srcp, dstp)

    tail = pl.pallas_call(
        _tail_body,
        in_specs=[pl.BlockSpec((2, PP, 128), lambda: (0, 0, 0)),
                  pl.BlockSpec((HP, PK), lambda: (0, 0)),
                  pl.BlockSpec((F1, 20), lambda: (0, 0)),
                  pl.BlockSpec((1, 20), lambda: (0, 0)),
                  pl.BlockSpec((1, 20), lambda: (0, 0)),
                  pl.BlockSpec((1, 20), lambda: (0, 0)),
                  pl.BlockSpec((20, 10), lambda: (0, 0)),
                  pl.BlockSpec((1, 10), lambda: (0, 0)),
                  pl.BlockSpec((10, 2), lambda: (0, 0)),
                  pl.BlockSpec((1, 2), lambda: (0, 0))],
        out_specs=pl.BlockSpec((G, 2), lambda: (0, 0)),
        out_shape=jax.ShapeDtypeStruct((G, 2), jnp.float32),
    )
    return tail(p2.reshape(NC, PP, 128), batch.reshape(HP, PK), W2,
                b2.reshape(1, 20), gamma.reshape(1, 20), beta.reshape(1, 20),
                Wf1, bf1.reshape(1, 10), Wf2, bf2.reshape(1, 2))
